# baseline (speedup 1.0000x reference)
import functools

import jax
import jax.numpy as jnp
from jax import lax
from jax.experimental import pallas as pl
from jax.experimental.pallas import tpu as pltpu
from jax.experimental.pallas import tpu_sc as plsc

_LANES = 16
_CHUNK_ROWS = 16


def kernel(x, pos_table):
    B, S, D = x.shape
    info = plsc.get_sparse_core_info()
    NC, NS = info.num_cores, info.num_subcores
    NW = NC * NS
    SPW = S // NW
    CH = _CHUNK_ROWS
    cps = SPW // CH
    CW = CH * D
    N = B * cps

    mesh = plsc.VectorSubcoreMesh(core_axis_name="c", subcore_axis_name="s")

    @functools.partial(
        pl.kernel,
        mesh=mesh,
        out_type=jax.ShapeDtypeStruct((B * S * D,), jnp.float32),
        scratch_types=[
            pltpu.VMEM((SPW * D,), jnp.float32),
            pltpu.VMEM((CW,), jnp.float32),
            pltpu.VMEM((CW,), jnp.float32),
            pltpu.SemaphoreType.DMA,
            pltpu.SemaphoreType.DMA,
            pltpu.SemaphoreType.DMA,
            pltpu.SemaphoreType.DMA,
        ],
    )
    def k(x_hbm, pos_hbm, out_hbm, pos_v, xbuf0, xbuf1, si0, si1, so0, so1):
        wid = lax.axis_index("s") * NC + lax.axis_index("c")
        s_base = wid * SPW
        pltpu.sync_copy(pos_hbm.at[pl.ds(s_base * D, SPW * D)], pos_v)

        bufs = (xbuf0, xbuf1)
        sin = (si0, si1)
        sout = (so0, so1)

        def off(ci):
            return ((ci // cps) * S + s_base + (ci % cps) * CH) * D

        def src(ci):
            return x_hbm.at[pl.ds(off(ci), CW)]

        def dst(ci):
            return out_hbm.at[pl.ds(off(ci), CW)]

        def compute(buf, ci):
            poff = (ci % cps) * CW

            @plsc.parallel_loop(0, CW, step=_LANES, unroll=8)
            def _(i):
                pv = pos_v[pl.ds(poff + i, _LANES)]
                plsc.addupdate(buf.at[pl.ds(i, _LANES)], pv)

        pltpu.async_copy(src(0), bufs[0], sin[0])

        def pair_body(g, carry):
            for b in range(2):
                ci = 2 * g + b

                @pl.when(ci >= 1)
                def _():
                    pltpu.make_async_copy(bufs[1 - b], dst(ci), sout[1 - b]).wait()

                @pl.when(ci + 1 < N)
                def _():
                    pltpu.async_copy(src(ci + 1), bufs[1 - b], sin[1 - b])

                pltpu.make_async_copy(src(ci), bufs[b], sin[b]).wait()
                compute(bufs[b], ci)
                pltpu.async_copy(bufs[b], dst(ci), sout[b])
            return carry

        lax.fori_loop(0, N // 2, pair_body, 0)
        pltpu.make_async_copy(bufs[1], dst(N - 1), sout[1]).wait()

    return k(x.reshape(-1), pos_table.reshape(-1)).reshape(B, S, D)


# speedup vs baseline: 2.0716x; 2.0716x over previous
import functools

import jax
import jax.numpy as jnp
from jax import lax
from jax.experimental import pallas as pl
from jax.experimental.pallas import tpu as pltpu
from jax.experimental.pallas import tpu_sc as plsc

_LANES = 16
_CHUNK_ROWS = 16


def kernel(x, pos_table):
    B, S, D = x.shape
    info = plsc.get_sparse_core_info()
    NC, NS = info.num_cores, info.num_subcores
    NW = NC * NS
    SPW = S // NW
    CH = _CHUNK_ROWS
    cps = SPW // CH
    CW = CH * D
    N = B * cps

    mesh = plsc.VectorSubcoreMesh(core_axis_name="c", subcore_axis_name="s")

    VECS = D // _LANES

    @functools.partial(
        pl.kernel,
        mesh=mesh,
        out_type=jax.ShapeDtypeStruct((B, S, D), jnp.float32),
        scratch_types=[
            pltpu.VMEM((SPW, D), jnp.float32),
            pltpu.VMEM((CH, D), jnp.float32),
            pltpu.VMEM((CH, D), jnp.float32),
            pltpu.SemaphoreType.DMA,
            pltpu.SemaphoreType.DMA,
            pltpu.SemaphoreType.DMA,
            pltpu.SemaphoreType.DMA,
        ],
    )
    def k(x_hbm, pos_hbm, out_hbm, pos_v, xbuf0, xbuf1, si0, si1, so0, so1):
        wid = lax.axis_index("s") * NC + lax.axis_index("c")
        s_base = wid * SPW
        pltpu.sync_copy(pos_hbm.at[pl.ds(s_base, SPW)], pos_v)

        bufs = (xbuf0, xbuf1)
        sin = (si0, si1)
        sout = (so0, so1)

        def src(ci):
            return x_hbm.at[ci // cps, pl.ds(s_base + (ci % cps) * CH, CH)]

        def dst(ci):
            return out_hbm.at[ci // cps, pl.ds(s_base + (ci % cps) * CH, CH)]

        def compute(buf, ci):
            prow0 = (ci % cps) * CH

            @plsc.parallel_loop(0, CH)
            def _(r):
                prow = prow0 + r
                for c in range(VECS):
                    pv = pos_v[prow, pl.ds(c * _LANES, _LANES)]
                    plsc.addupdate(buf.at[r, pl.ds(c * _LANES, _LANES)], pv)

        pltpu.async_copy(src(0), bufs[0], sin[0])

        def pair_body(g, carry):
            for b in range(2):
                ci = 2 * g + b

                @pl.when(ci >= 1)
                def _():
                    pltpu.make_async_copy(bufs[1 - b], dst(ci), sout[1 - b]).wait()

                @pl.when(ci + 1 < N)
                def _():
                    pltpu.async_copy(src(ci + 1), bufs[1 - b], sin[1 - b])

                pltpu.make_async_copy(src(ci), bufs[b], sin[b]).wait()
                compute(bufs[b], ci)
                pltpu.async_copy(bufs[b], dst(ci), sout[b])
            return carry

        lax.fori_loop(0, N // 2, pair_body, 0)
        pltpu.make_async_copy(bufs[1], dst(N - 1), sout[1]).wait()

    return k(x, pos_table)
